# Initial kernel scaffold; baseline (speedup 1.0000x reference)
import jax
import jax.numpy as jnp
from jax.experimental import pallas as pl


def kernel(x, ei_g1_pos, ei_g2_pos, ei_g1_neg, ei_g2_neg, Wp0, asp0, adp0, bp0, Wp1, asp1, adp1, bp1, Wn0, asn0, adn0, bn0, Wn1, asn1, adn1, bn1, prelu_a):
    raise NotImplementedError("write your pallas kernel here")


# speedup vs baseline: 17.4694x; 17.4694x over previous
import jax
import jax.numpy as jnp
from jax import lax
from jax.experimental import pallas as pl
from jax.experimental.pallas import tpu as pltpu
from jax.experimental.pallas import tpu_sc as plsc

N = 10000
D = 128
E = 320000
E2 = E + N
NT = 16
TB = 128
CB = 162
CHUNK = CB * TB
E2P = CHUNK * NT
NP_ = 10240
GHOST = NP_ - 2
APW = 20480
BN = 400


def _dense_body(y_ref, w_ref, a_ref, h_ref, ap_ref):
    xb = y_ref[0]
    w = w_ref[0]
    h = lax.dot_general(xb, w, (((1,), (1,)), ((), ())),
                        preferred_element_type=jnp.float32)
    h_ref[0] = h
    ap_ref[0] = lax.dot_general(h, a_ref[0], (((1,), (0,)), ((), ())),
                                preferred_element_type=jnp.float32)


def _dense(y3, wst, ast):
    ymap = lambda b, i: (b, i, 0)
    return pl.pallas_call(
        _dense_body,
        grid=(4, N // BN),
        in_specs=[pl.BlockSpec((1, BN, D), ymap),
                  pl.BlockSpec((1, D, D), lambda b, i: (b, 0, 0)),
                  pl.BlockSpec((1, D, 2), lambda b, i: (b, 0, 0))],
        out_specs=[pl.BlockSpec((1, BN, D), lambda b, i: (b, i, 0)),
                   pl.BlockSpec((1, BN, 2), lambda b, i: (b, i, 0))],
        out_shape=[jax.ShapeDtypeStruct((4, N, D), jnp.float32),
                   jax.ShapeDtypeStruct((4, N, 2), jnp.float32)],
    )(y3, wst, ast)


def _epi_stacked_body(r0, r1, r2, r3, den_ref, b_ref, a_ref, y_ref):
    a = a_ref[0, 0]
    for j in range(4):
        raw = jnp.concatenate([r0[j], r1[j], r2[j], r3[j]], axis=1)
        z = raw / (den_ref[j] + 1e-16) + b_ref[j][None, :]
        y_ref[j] = jnp.where(z >= 0.0, z, a * z)


def _epilogue_stacked(rq, den3, bst, pa):
    qspec = pl.BlockSpec((4, BN, D // 4), lambda i: (0, i, 0))
    return pl.pallas_call(
        _epi_stacked_body,
        grid=(N // BN,),
        in_specs=[qspec] * 4 + [
            pl.BlockSpec((4, BN, 1), lambda i: (0, i, 0)),
            pl.BlockSpec((4, D), lambda i: (0, 0)),
            pl.BlockSpec((1, 1), lambda i: (0, 0)),
        ],
        out_specs=[pl.BlockSpec((4, BN, D), lambda i: (0, i, 0))],
        out_shape=[jax.ShapeDtypeStruct((4, N, D), jnp.float32)],
    )(*rq, den3, bst, pa)[0]


QW = D // 4


def _edge_body(h0, h1, h2, h3, ap_hbm, srcs, dsts, rowidx_hbm,
               rq0, rq1, rq2, rq3, den_o,
               src_t, dst_t, ap_t, ex_t, denloc, rowidx_t, gbuf, sbuf, zbuf,
               raw_s, den_s, sem):
    c = lax.axis_index("c")
    s = lax.axis_index("s")

    pltpu.sync_copy(rowidx_hbm, rowidx_t)

    def _zz(i, _):
        for f in range(2):
            zbuf[i, pl.ds(f * 16, 16)] = jnp.zeros((16,), jnp.float32)
        return 0
    lax.fori_loop(0, 160, _zz, 0)

    def _zero_raw():
        for q in range(4):
            pltpu.sync_copy(zbuf, raw_s.at[pl.ds(s * 640 + q * 160, 160)])

    for bi in range(4):
        pltpu.sync_copy(srcs.at[bi, s], src_t)
        pltpu.sync_copy(dsts.at[bi, s], dst_t)
        pltpu.sync_copy(ap_hbm.at[bi], ap_t)

        def _zd(i, _):
            denloc[i] = jnp.zeros((16,), jnp.float32)
            return 0
        lax.fori_loop(0, 640, _zd, 0)
        _zero_raw()
        pltpu.sync_copy(denloc.at[pl.ds(s * 40, 40)], den_s.at[pl.ds(s * 40, 40)])

        boff = bi * N

        def _p1(j, _):
            base = j * TB
            for g in range(8):
                off = base + g * 16
                sv = src_t[pl.ds(off, 16)]
                dv = dst_t[j, pl.ds(g * 16, 16)]
                su = sv - boff
                av = plsc.load_gather(ap_t, [su + su])
                bv = plsc.load_gather(ap_t, [dv + dv + 1])
                al = av + bv
                al = jnp.where(al >= 0.0, al, 0.2 * al)
                e = jnp.exp(al)
                ex_t[pl.ds(off, 16)] = e
                plsc.addupdate_scatter(denloc, [dv >> 4, dv & 15], e)
            return 0
        lax.fori_loop(0, CB, _p1, 0)

        plsc.subcore_barrier()

        for q in range(5):
            pltpu.sync_copy(denloc.at[pl.ds(q * 128, 128)],
                            den_s.at[rowidx_t.at[q]], add=True)
        plsc.subcore_barrier()

        @pl.when(c == 0)
        def _():
            pltpu.sync_copy(den_s.at[pl.ds(s * 40, 40)],
                            den_o.at[bi, pl.ds(s * 40, 40)])

        def _p3(h_hbm):
            def _body(t, _):
                pltpu.async_copy(h_hbm.at[src_t.at[pl.ds(t * TB, TB)]],
                                 gbuf, sem).wait()

                def _sc16(r16, _2):
                    ev16 = ex_t[pl.ds(t * TB + r16 * 16, 16)]
                    for rr in range(16):
                        evb = jnp.broadcast_to(ev16[rr], (16,))
                        r = r16 * 16 + rr
                        for f in range(2):
                            sl = pl.ds(f * 16, 16)
                            sbuf[r, sl] = gbuf[r, sl] * evb
                    return 0
                lax.fori_loop(0, 8, _sc16, 0)
                pltpu.sync_copy(sbuf, raw_s.at[dst_t.at[t]], add=True)
                return 0
            lax.fori_loop(0, CB, _body, 0)

        def _wout(rq):
            pltpu.sync_copy(raw_s.at[pl.ds(s * 640, 640)],
                            rq.at[bi, pl.ds(s * 640, 640)])

        @pl.when(c == 0)
        def _():
            _p3(h0)

        @pl.when(c == 1)
        def _():
            _p3(h2)

        plsc.subcore_barrier()

        @pl.when(c == 0)
        def _():
            _wout(rq0)

        @pl.when(c == 1)
        def _():
            _wout(rq2)

        plsc.subcore_barrier()
        _zero_raw()
        plsc.subcore_barrier()

        @pl.when(c == 0)
        def _():
            _p3(h1)

        @pl.when(c == 1)
        def _():
            _p3(h3)

        plsc.subcore_barrier()

        @pl.when(c == 0)
        def _():
            _wout(rq1)

        @pl.when(c == 1)
        def _():
            _wout(rq3)

        if bi < 3:
            plsc.subcore_barrier()


def _edge_kernel(h0, h1, h2, h3, ap_pad, srcs, dsts, rowidx):
    return pl.kernel(
        _edge_body,
        mesh=plsc.VectorSubcoreMesh(core_axis_name="c", subcore_axis_name="s"),
        compiler_params=pltpu.CompilerParams(needs_layout_passes=False,
                                             use_tc_tiling_on_sc=False),
        out_type=[jax.ShapeDtypeStruct((4, NP_, QW), jnp.float32)] * 4
        + [jax.ShapeDtypeStruct((4, 640, 16), jnp.float32)],
        scratch_types=[
            pltpu.VMEM((CHUNK,), jnp.int32),
            pltpu.VMEM((CB, TB), jnp.int32),
            pltpu.VMEM((APW,), jnp.float32),
            pltpu.VMEM((CHUNK,), jnp.float32),
            pltpu.VMEM((640, 16), jnp.float32),
            pltpu.VMEM((5, 128), jnp.int32),
            pltpu.VMEM((TB, QW), jnp.float32),
            pltpu.VMEM((TB, QW), jnp.float32),
            pltpu.VMEM((160, QW), jnp.float32),
            pltpu.VMEM_SHARED((NP_, QW), jnp.float32),
            pltpu.VMEM_SHARED((640, 16), jnp.float32),
            pltpu.SemaphoreType.DMA,
        ],
    )(h0, h1, h2, h3, ap_pad, srcs, dsts, rowidx)


def kernel(x, ei_g1_pos, ei_g2_pos, ei_g1_neg, ei_g2_neg,
           Wp0, asp0, adp0, bp0, Wp1, asp1, adp1, bp1,
           Wn0, asn0, adn0, bn0, Wn1, asn1, adn1, bn1,
           prelu_a):
    loop = jnp.arange(N, dtype=jnp.int32)
    srcs, dsts = [], []
    for b, ei in enumerate((ei_g1_pos, ei_g2_pos, ei_g1_neg, ei_g2_neg)):
        s_full = jnp.concatenate(
            [ei[0], loop, jnp.zeros((E2P - E2,), jnp.int32)]) + b * N
        d_full = jnp.concatenate(
            [ei[1], loop, jnp.full((E2P - E2,), GHOST, jnp.int32)])
        srcs.append(s_full.reshape(NT, CHUNK))
        dsts.append(d_full.reshape(NT, CB, TB))
    srcs = jnp.stack(srcs)
    dsts = jnp.stack(dsts)
    rowidx = jnp.arange(640, dtype=jnp.int32).reshape(5, 128)

    Wst = jnp.stack([jnp.stack([Wp0, Wp0, Wn0, Wn0]),
                     jnp.stack([Wp1, Wp1, Wn1, Wn1])])
    a0p = jnp.stack([asp0, adp0], axis=1)
    a0n = jnp.stack([asn0, adn0], axis=1)
    a1p = jnp.stack([asp1, adp1], axis=1)
    a1n = jnp.stack([asn1, adn1], axis=1)
    Ast = jnp.stack([jnp.stack([a0p, a0p, a0n, a0n]),
                     jnp.stack([a1p, a1p, a1n, a1n])])
    Bst = jnp.stack([jnp.stack([bp0, bp0, bn0, bn0]),
                     jnp.stack([bp1, bp1, bn1, bn1])])
    pa = prelu_a.reshape(1, 1)

    def layer_step(y, wab):
        w4, a4, b4 = wab
        h, ap = _dense(y, w4, a4)
        ap_pad = jnp.pad(ap.reshape(4, 2 * N), ((0, 0), (0, APW - 2 * N)))
        hf = h.reshape(4 * N, D)
        out = _edge_kernel(hf[:, 0:32], hf[:, 32:64], hf[:, 64:96],
                           hf[:, 96:128], ap_pad, srcs, dsts, rowidx)
        den3 = out[4].reshape(4, NP_, 1)
        return _epilogue_stacked(out[:4], den3, b4, pa), 0

    y0 = jnp.broadcast_to(x[None], (4, N, D))
    yfin, _ = lax.scan(layer_step, y0, (Wst, Ast, Bst))
    return (yfin[0], yfin[1], yfin[2], yfin[3])


# speedup vs baseline: 25.6090x; 1.4659x over previous
import jax
import jax.numpy as jnp
from jax import lax
from jax.experimental import pallas as pl
from jax.experimental.pallas import tpu as pltpu
from jax.experimental.pallas import tpu_sc as plsc

N = 10000
D = 128
E = 320000
E2 = E + N
NT = 16
TB = 128
CB = 162
CHUNK = CB * TB
E2P = CHUNK * NT
NP_ = 10048
NPD = 10240
GHOST = NP_ - 2
APW = 20480
BN = 400


def _dense_body(y_ref, w_ref, a_ref, h_ref, ap_ref):
    xb = y_ref[0]
    w = w_ref[0]
    h = lax.dot_general(xb, w, (((1,), (1,)), ((), ())),
                        preferred_element_type=jnp.float32)
    h_ref[0] = h
    ap_ref[0] = lax.dot_general(h, a_ref[0], (((1,), (0,)), ((), ())),
                                preferred_element_type=jnp.float32)


def _dense(y3, wst, ast):
    ymap = lambda b, i: (b, i, 0)
    return pl.pallas_call(
        _dense_body,
        grid=(4, N // BN),
        in_specs=[pl.BlockSpec((1, BN, D), ymap),
                  pl.BlockSpec((1, D, D), lambda b, i: (b, 0, 0)),
                  pl.BlockSpec((1, D, 2), lambda b, i: (b, 0, 0))],
        out_specs=[pl.BlockSpec((1, BN, D), lambda b, i: (b, i, 0)),
                   pl.BlockSpec((1, BN, 2), lambda b, i: (b, i, 0))],
        out_shape=[jax.ShapeDtypeStruct((4, N, D), jnp.float32),
                   jax.ShapeDtypeStruct((4, N, 2), jnp.float32)],
    )(y3, wst, ast)


def _epi_stacked_body(r0, r1, r2, r3, den_ref, b_ref, a_ref, y_ref):
    a = a_ref[0, 0]
    for j in range(4):
        raw = jnp.concatenate([r0[j], r1[j], r2[j], r3[j]], axis=1)
        z = raw / (den_ref[j] + 1e-16) + b_ref[j][None, :]
        y_ref[j] = jnp.where(z >= 0.0, z, a * z)


def _epilogue_stacked(rq, den3, bst, pa):
    qspec = pl.BlockSpec((4, BN, D // 4), lambda i: (0, i, 0))
    return pl.pallas_call(
        _epi_stacked_body,
        grid=(N // BN,),
        in_specs=[qspec] * 4 + [
            pl.BlockSpec((4, BN, 1), lambda i: (0, i, 0)),
            pl.BlockSpec((4, D), lambda i: (0, 0)),
            pl.BlockSpec((1, 1), lambda i: (0, 0)),
        ],
        out_specs=[pl.BlockSpec((4, BN, D), lambda i: (0, i, 0))],
        out_shape=[jax.ShapeDtypeStruct((4, N, D), jnp.float32)],
    )(*rq, den3, bst, pa)[0]


QW = D // 4


def _edge_body(h0, h1, h2, h3, ap_hbm, srcs, dsts,
               rq0, rq1, rq2, rq3, den_o,
               src_t, dst_t, ap_t, ex_t, denloc,
               gbuf, gbuf2, sbuf, zbuf,
               raw_s, gsem, gsem2):
    c = lax.axis_index("c")
    s = lax.axis_index("s")

    def _zz(i, _):
        for f in range(2):
            zbuf[i, pl.ds(f * 16, 16)] = jnp.zeros((16,), jnp.float32)
        return 0
    lax.fori_loop(0, 157, _zz, 0)

    def _zero_raw():
        for q in range(4):
            pltpu.sync_copy(zbuf, raw_s.at[pl.ds(s * 628 + q * 157, 157)])

    for bi in range(4):
        pltpu.sync_copy(srcs.at[bi, s], src_t)
        pltpu.sync_copy(dsts.at[bi, s], dst_t)
        pltpu.sync_copy(ap_hbm.at[bi], ap_t)

        def _zd(i, _):
            denloc[i] = jnp.zeros((16,), jnp.float32)
            return 0
        lax.fori_loop(0, 640, _zd, 0)
        _zero_raw()

        boff = bi * N

        def _p1(j, _):
            base = j * TB
            for g in range(8):
                off = base + g * 16
                sv = src_t[pl.ds(off, 16)]
                dv = dst_t[j, pl.ds(g * 16, 16)]
                su = sv - boff
                av = plsc.load_gather(ap_t, [su + su])
                bv = plsc.load_gather(ap_t, [dv + dv + 1])
                al = av + bv
                al = jnp.where(al >= 0.0, al, 0.2 * al)
                e = jnp.exp(al)
                ex_t[pl.ds(off, 16)] = e
                plsc.addupdate_scatter(denloc, [dv >> 4, dv & 15], e)
            return 0
        lax.fori_loop(0, CB, _p1, 0)

        @pl.when(c == 0)
        def _():
            pltpu.sync_copy(denloc, den_o.at[bi, s])

        plsc.subcore_barrier()

        def _p3(h_hbm):
            gb = (gbuf, gbuf2)
            gs = (gsem, gsem2)

            def _start_g(t, k):
                pltpu.async_copy(h_hbm.at[src_t.at[pl.ds(t * TB, TB)]],
                                 gb[k], gs[k])

            def _wait_g(t, k):
                pltpu.make_async_copy(h_hbm.at[src_t.at[pl.ds(t * TB, TB)]],
                                      gb[k], gs[k]).wait()

            def _scale_store(t, k):
                def _sc16(r16, _2):
                    ev16 = ex_t[pl.ds(t * TB + r16 * 16, 16)]
                    for rr in range(16):
                        evb = jnp.broadcast_to(ev16[rr], (16,))
                        r = r16 * 16 + rr
                        for f in range(2):
                            sl = pl.ds(f * 16, 16)
                            sbuf[r, sl] = gb[k][r, sl] * evb
                    return 0
                lax.fori_loop(0, 8, _sc16, 0)
                pltpu.sync_copy(sbuf, raw_s.at[dst_t.at[t]], add=True)

            _start_g(0, 0)

            def _pair(p, _):
                t0 = 2 * p
                t1 = t0 + 1
                _start_g(t1, 1)
                _wait_g(t0, 0)
                _scale_store(t0, 0)

                @pl.when(p < CB // 2 - 1)
                def _():
                    _start_g(t0 + 2, 0)
                _wait_g(t1, 1)
                _scale_store(t1, 1)
                return 0
            lax.fori_loop(0, CB // 2, _pair, 0)

        def _wout(rq):
            pltpu.sync_copy(raw_s.at[pl.ds(s * 628, 628)],
                            rq.at[bi, pl.ds(s * 628, 628)])

        @pl.when(c == 0)
        def _():
            _p3(h0)

        @pl.when(c == 1)
        def _():
            _p3(h2)

        plsc.subcore_barrier()

        @pl.when(c == 0)
        def _():
            _wout(rq0)

        @pl.when(c == 1)
        def _():
            _wout(rq2)

        plsc.subcore_barrier()
        _zero_raw()
        plsc.subcore_barrier()

        @pl.when(c == 0)
        def _():
            _p3(h1)

        @pl.when(c == 1)
        def _():
            _p3(h3)

        plsc.subcore_barrier()

        @pl.when(c == 0)
        def _():
            _wout(rq1)

        @pl.when(c == 1)
        def _():
            _wout(rq3)

        if bi < 3:
            plsc.subcore_barrier()


def _edge_kernel(h0, h1, h2, h3, ap_pad, srcs, dsts):
    return pl.kernel(
        _edge_body,
        mesh=plsc.VectorSubcoreMesh(core_axis_name="c", subcore_axis_name="s"),
        compiler_params=pltpu.CompilerParams(needs_layout_passes=False,
                                             use_tc_tiling_on_sc=False),
        out_type=[jax.ShapeDtypeStruct((4, NP_, QW), jnp.float32)] * 4
        + [jax.ShapeDtypeStruct((4, NT, 640, 16), jnp.float32)],
        scratch_types=[
            pltpu.VMEM((CHUNK,), jnp.int32),
            pltpu.VMEM((CB, TB), jnp.int32),
            pltpu.VMEM((APW,), jnp.float32),
            pltpu.VMEM((CHUNK,), jnp.float32),
            pltpu.VMEM((640, 16), jnp.float32),
            pltpu.VMEM((TB, QW), jnp.float32),
            pltpu.VMEM((TB, QW), jnp.float32),
            pltpu.VMEM((TB, QW), jnp.float32),
            pltpu.VMEM((157, QW), jnp.float32),
            pltpu.VMEM_SHARED((NP_, QW), jnp.float32),
            pltpu.SemaphoreType.DMA,
            pltpu.SemaphoreType.DMA,
        ],
    )(h0, h1, h2, h3, ap_pad, srcs, dsts)


def kernel(x, ei_g1_pos, ei_g2_pos, ei_g1_neg, ei_g2_neg,
           Wp0, asp0, adp0, bp0, Wp1, asp1, adp1, bp1,
           Wn0, asn0, adn0, bn0, Wn1, asn1, adn1, bn1,
           prelu_a):
    loop = jnp.arange(N, dtype=jnp.int32)
    srcs, dsts = [], []
    for b, ei in enumerate((ei_g1_pos, ei_g2_pos, ei_g1_neg, ei_g2_neg)):
        s_full = jnp.concatenate(
            [ei[0], loop, jnp.zeros((E2P - E2,), jnp.int32)]) + b * N
        d_full = jnp.concatenate(
            [ei[1], loop, jnp.full((E2P - E2,), GHOST, jnp.int32)])
        srcs.append(s_full.reshape(NT, CHUNK))
        dsts.append(d_full.reshape(NT, CB, TB))
    srcs = jnp.stack(srcs)
    dsts = jnp.stack(dsts)

    Wst = jnp.stack([jnp.stack([Wp0, Wp0, Wn0, Wn0]),
                     jnp.stack([Wp1, Wp1, Wn1, Wn1])])
    a0p = jnp.stack([asp0, adp0], axis=1)
    a0n = jnp.stack([asn0, adn0], axis=1)
    a1p = jnp.stack([asp1, adp1], axis=1)
    a1n = jnp.stack([asn1, adn1], axis=1)
    Ast = jnp.stack([jnp.stack([a0p, a0p, a0n, a0n]),
                     jnp.stack([a1p, a1p, a1n, a1n])])
    Bst = jnp.stack([jnp.stack([bp0, bp0, bn0, bn0]),
                     jnp.stack([bp1, bp1, bn1, bn1])])
    pa = prelu_a.reshape(1, 1)

    def layer_step(y, wab):
        w4, a4, b4 = wab
        h, ap = _dense(y, w4, a4)
        ap_pad = jnp.pad(ap.reshape(4, 2 * N), ((0, 0), (0, APW - 2 * N)))
        hf = h.reshape(4 * N, D)
        out = _edge_kernel(hf[:, 0:32], hf[:, 32:64], hf[:, 64:96],
                           hf[:, 96:128], ap_pad, srcs, dsts)
        den3 = out[4].sum(axis=1).reshape(4, NPD, 1)
        return _epilogue_stacked(out[:4], den3, b4, pa), 0

    y0 = jnp.broadcast_to(x[None], (4, N, D))
    yfin, _ = lax.scan(layer_step, y0, (Wst, Ast, Bst))
    return (yfin[0], yfin[1], yfin[2], yfin[3])


# speedup vs baseline: 26.1554x; 1.0213x over previous
import jax
import jax.numpy as jnp
from jax import lax
from jax.experimental import pallas as pl
from jax.experimental.pallas import tpu as pltpu
from jax.experimental.pallas import tpu_sc as plsc

N = 10000
D = 128
E = 320000
E2 = E + N
NT = 16
TB = 128
CB = 162
CHUNK = CB * TB
E2P = CHUNK * NT
NP_ = 10048
NPD = 10240
GHOST = NP_ - 2
APW = 20480
BN = 400
QW = D // 4


def _dense_body(y_ref, w_ref, a_ref, h0r, h1r, h2r, h3r, ap_ref):
    xb = y_ref[0]
    w = w_ref[0]
    h = lax.dot_general(xb, w, (((1,), (1,)), ((), ())),
                        preferred_element_type=jnp.float32)
    for k, hr in enumerate((h0r, h1r, h2r, h3r)):
        hr[0] = h[:, k * QW:(k + 1) * QW]
    ap_ref[0] = lax.dot_general(h, a_ref[0], (((1,), (0,)), ((), ())),
                                preferred_element_type=jnp.float32)


def _dense(y3, wst, ast):
    qspec = pl.BlockSpec((1, BN, QW), lambda b, i: (b, i, 0))
    return pl.pallas_call(
        _dense_body,
        grid=(4, N // BN),
        in_specs=[pl.BlockSpec((1, BN, D), lambda b, i: (b, i, 0)),
                  pl.BlockSpec((1, D, D), lambda b, i: (b, 0, 0)),
                  pl.BlockSpec((1, D, 2), lambda b, i: (b, 0, 0))],
        out_specs=[qspec] * 4
        + [pl.BlockSpec((1, BN, 2), lambda b, i: (b, i, 0))],
        out_shape=[jax.ShapeDtypeStruct((4, N, QW), jnp.float32)] * 4
        + [jax.ShapeDtypeStruct((4, N, 2), jnp.float32)],
    )(y3, wst, ast)


def _epi_stacked_body(r0, r1, r2, r3, den_ref, b_ref, a_ref, y_ref):
    a = a_ref[0, 0]
    for j in range(4):
        raw = jnp.concatenate([r0[j], r1[j], r2[j], r3[j]], axis=1)
        z = raw / (den_ref[j] + 1e-16) + b_ref[j][None, :]
        y_ref[j] = jnp.where(z >= 0.0, z, a * z)


def _epilogue_stacked(rq, den3, bst, pa):
    qspec = pl.BlockSpec((4, BN, QW), lambda i: (0, i, 0))
    return pl.pallas_call(
        _epi_stacked_body,
        grid=(N // BN,),
        in_specs=[qspec] * 4 + [
            pl.BlockSpec((4, BN, 1), lambda i: (0, i, 0)),
            pl.BlockSpec((4, D), lambda i: (0, 0)),
            pl.BlockSpec((1, 1), lambda i: (0, 0)),
        ],
        out_specs=[pl.BlockSpec((4, BN, D), lambda i: (0, i, 0))],
        out_shape=[jax.ShapeDtypeStruct((4, N, D), jnp.float32)],
    )(*rq, den3, bst, pa)[0]


def _edge_body(h0, h1, h2, h3, ap_hbm, srcs, dsts,
               rq0, rq1, rq2, rq3, den_o,
               src_t, dst_t, ap_t, ex_t, denloc,
               gbuf, gbuf2, sbuf, zbuf,
               raw_s, gsem, gsem2):
    c = lax.axis_index("c")
    s = lax.axis_index("s")

    def _zz(i, _):
        for f in range(2):
            zbuf[i, pl.ds(f * 16, 16)] = jnp.zeros((16,), jnp.float32)
        return 0
    lax.fori_loop(0, 157, _zz, 0)

    def _zero_raw():
        for q in range(4):
            pltpu.sync_copy(zbuf, raw_s.at[pl.ds(s * 628 + q * 157, 157)])

    def _branch(bi, _):
        pltpu.sync_copy(srcs.at[bi, s], src_t)
        pltpu.sync_copy(dsts.at[bi, s], dst_t)
        pltpu.sync_copy(ap_hbm.at[bi], ap_t)

        def _zd(i, _2):
            denloc[i] = jnp.zeros((16,), jnp.float32)
            return 0
        lax.fori_loop(0, 640, _zd, 0)
        _zero_raw()

        boff = bi * N

        def _p1(j, _2):
            base = j * TB
            for g in range(8):
                off = base + g * 16
                sv = src_t[pl.ds(off, 16)]
                dv = dst_t[j, pl.ds(g * 16, 16)]
                su = sv - boff
                av = plsc.load_gather(ap_t, [su + su])
                bv = plsc.load_gather(ap_t, [dv + dv + 1])
                al = av + bv
                al = jnp.where(al >= 0.0, al, 0.2 * al)
                e = jnp.exp(al)
                ex_t[pl.ds(off, 16)] = e
                plsc.addupdate_scatter(denloc, [dv >> 4, dv & 15], e)
            return 0
        lax.fori_loop(0, CB, _p1, 0)

        @pl.when(c == 0)
        def _():
            pltpu.sync_copy(denloc, den_o.at[bi, s])

        plsc.subcore_barrier()

        def _p3(h_hbm):
            gb = (gbuf, gbuf2)
            gs = (gsem, gsem2)

            def _start_g(t, k):
                pltpu.async_copy(h_hbm.at[src_t.at[pl.ds(t * TB, TB)]],
                                 gb[k], gs[k])

            def _wait_g(t, k):
                pltpu.make_async_copy(h_hbm.at[src_t.at[pl.ds(t * TB, TB)]],
                                      gb[k], gs[k]).wait()

            def _scale_store(t, k):
                def _sc16(r16, _2):
                    ev16 = ex_t[pl.ds(t * TB + r16 * 16, 16)]
                    for rr in range(16):
                        evb = jnp.broadcast_to(ev16[rr], (16,))
                        r = r16 * 16 + rr
                        for f in range(2):
                            sl = pl.ds(f * 16, 16)
                            sbuf[r, sl] = gb[k][r, sl] * evb
                    return 0
                lax.fori_loop(0, 8, _sc16, 0)
                pltpu.sync_copy(sbuf, raw_s.at[dst_t.at[t]], add=True)

            _start_g(0, 0)

            def _pair(p, _2):
                t0 = 2 * p
                t1 = t0 + 1
                _start_g(t1, 1)
                _wait_g(t0, 0)
                _scale_store(t0, 0)

                @pl.when(p < CB // 2 - 1)
                def _():
                    _start_g(t0 + 2, 0)
                _wait_g(t1, 1)
                _scale_store(t1, 1)
                return 0
            lax.fori_loop(0, CB // 2, _pair, 0)

        def _wout(rq):
            pltpu.sync_copy(raw_s.at[pl.ds(s * 628, 628)],
                            rq.at[bi, pl.ds(s * 628, 628)])

        @pl.when(c == 0)
        def _():
            _p3(h0)

        @pl.when(c == 1)
        def _():
            _p3(h2)

        plsc.subcore_barrier()

        @pl.when(c == 0)
        def _():
            _wout(rq0)

        @pl.when(c == 1)
        def _():
            _wout(rq2)

        plsc.subcore_barrier()
        _zero_raw()
        plsc.subcore_barrier()

        @pl.when(c == 0)
        def _():
            _p3(h1)

        @pl.when(c == 1)
        def _():
            _p3(h3)

        plsc.subcore_barrier()

        @pl.when(c == 0)
        def _():
            _wout(rq1)

        @pl.when(c == 1)
        def _():
            _wout(rq3)

        plsc.subcore_barrier()
        return 0

    lax.fori_loop(0, 4, _branch, 0)


def _edge_kernel(h0, h1, h2, h3, ap_pad, srcs, dsts):
    return pl.kernel(
        _edge_body,
        mesh=plsc.VectorSubcoreMesh(core_axis_name="c", subcore_axis_name="s"),
        compiler_params=pltpu.CompilerParams(needs_layout_passes=False,
                                             use_tc_tiling_on_sc=False),
        out_type=[jax.ShapeDtypeStruct((4, NP_, QW), jnp.float32)] * 4
        + [jax.ShapeDtypeStruct((4, NT, 640, 16), jnp.float32)],
        scratch_types=[
            pltpu.VMEM((CHUNK,), jnp.int32),
            pltpu.VMEM((CB, TB), jnp.int32),
            pltpu.VMEM((APW,), jnp.float32),
            pltpu.VMEM((CHUNK,), jnp.float32),
            pltpu.VMEM((640, 16), jnp.float32),
            pltpu.VMEM((TB, QW), jnp.float32),
            pltpu.VMEM((TB, QW), jnp.float32),
            pltpu.VMEM((TB, QW), jnp.float32),
            pltpu.VMEM((157, QW), jnp.float32),
            pltpu.VMEM_SHARED((NP_, QW), jnp.float32),
            pltpu.SemaphoreType.DMA,
            pltpu.SemaphoreType.DMA,
        ],
    )(h0, h1, h2, h3, ap_pad, srcs, dsts)


def kernel(x, ei_g1_pos, ei_g2_pos, ei_g1_neg, ei_g2_neg,
           Wp0, asp0, adp0, bp0, Wp1, asp1, adp1, bp1,
           Wn0, asn0, adn0, bn0, Wn1, asn1, adn1, bn1,
           prelu_a):
    loop = jnp.arange(N, dtype=jnp.int32)
    srcs, dsts = [], []
    for b, ei in enumerate((ei_g1_pos, ei_g2_pos, ei_g1_neg, ei_g2_neg)):
        s_full = jnp.concatenate(
            [ei[0], loop, jnp.zeros((E2P - E2,), jnp.int32)]) + b * N
        d_full = jnp.concatenate(
            [ei[1], loop, jnp.full((E2P - E2,), GHOST, jnp.int32)])
        srcs.append(s_full.reshape(NT, CHUNK))
        dsts.append(d_full.reshape(NT, CB, TB))
    srcs = jnp.stack(srcs)
    dsts = jnp.stack(dsts)

    Wst = jnp.stack([jnp.stack([Wp0, Wp0, Wn0, Wn0]),
                     jnp.stack([Wp1, Wp1, Wn1, Wn1])])
    a0p = jnp.stack([asp0, adp0], axis=1)
    a0n = jnp.stack([asn0, adn0], axis=1)
    a1p = jnp.stack([asp1, adp1], axis=1)
    a1n = jnp.stack([asn1, adn1], axis=1)
    Ast = jnp.stack([jnp.stack([a0p, a0p, a0n, a0n]),
                     jnp.stack([a1p, a1p, a1n, a1n])])
    Bst = jnp.stack([jnp.stack([bp0, bp0, bn0, bn0]),
                     jnp.stack([bp1, bp1, bn1, bn1])])
    pa = prelu_a.reshape(1, 1)

    def layer_step(y, wab):
        w4, a4, b4 = wab
        h0, h1, h2, h3, ap = _dense(y, w4, a4)
        ap_pad = jnp.pad(ap.reshape(4, 2 * N), ((0, 0), (0, APW - 2 * N)))
        out = _edge_kernel(h0.reshape(4 * N, QW), h1.reshape(4 * N, QW),
                           h2.reshape(4 * N, QW), h3.reshape(4 * N, QW),
                           ap_pad, srcs, dsts)
        den3 = out[4].sum(axis=1).reshape(4, NPD, 1)
        return _epilogue_stacked(out[:4], den3, b4, pa), 0

    y0 = jnp.broadcast_to(x[None], (4, N, D))
    yfin, _ = lax.scan(layer_step, y0, (Wst, Ast, Bst))
    return (yfin[0], yfin[1], yfin[2], yfin[3])


# speedup vs baseline: 37.5971x; 1.4375x over previous
import jax
import jax.numpy as jnp
from jax import lax
from jax.experimental import pallas as pl
from jax.experimental.pallas import tpu as pltpu
from jax.experimental.pallas import tpu_sc as plsc

N = 10000
D = 128
E = 320000
E2 = E + N
NT = 16
TB = 128
CB = 162
CHUNK = CB * TB
E2P = CHUNK * NT
NP_ = 10048
NPD = 10240
GHOST = NP_ - 2
APW = 20480
BN = 400
QW = D // 4
HW = D // 2


def _dense_body(y_ref, w_ref, a_ref, h0r, h1r, ap_ref):
    xb = y_ref[0]
    w = w_ref[0]
    h = lax.dot_general(xb, w, (((1,), (1,)), ((), ())),
                        preferred_element_type=jnp.float32)
    hb = h.astype(jnp.bfloat16)
    h0r[0] = hb[:, :HW]
    h1r[0] = hb[:, HW:]
    ap_ref[0] = lax.dot_general(h, a_ref[0], (((1,), (0,)), ((), ())),
                                preferred_element_type=jnp.float32)


def _dense(y3, wst, ast):
    qspec = pl.BlockSpec((1, BN, HW), lambda b, i: (b, i, 0))
    return pl.pallas_call(
        _dense_body,
        grid=(4, N // BN),
        in_specs=[pl.BlockSpec((1, BN, D), lambda b, i: (b, i, 0)),
                  pl.BlockSpec((1, D, D), lambda b, i: (b, 0, 0)),
                  pl.BlockSpec((1, D, 2), lambda b, i: (b, 0, 0))],
        out_specs=[qspec] * 2
        + [pl.BlockSpec((1, BN, 2), lambda b, i: (b, i, 0))],
        out_shape=[jax.ShapeDtypeStruct((4, N, HW), jnp.bfloat16)] * 2
        + [jax.ShapeDtypeStruct((4, N, 2), jnp.float32)],
    )(y3, wst, ast)


def _epi_stacked_body(r0, r1, den_ref, b_ref, a_ref, y_ref):
    a = a_ref[0, 0]
    for j in range(4):
        raw = jnp.concatenate([r0[j], r1[j]], axis=1).astype(jnp.float32)
        z = raw / (den_ref[j] + 1e-16) + b_ref[j][None, :]
        y_ref[j] = jnp.where(z >= 0.0, z, a * z)


def _epilogue_stacked(rq, den3, bst, pa):
    qspec = pl.BlockSpec((4, BN, HW), lambda i: (0, i, 0))
    return pl.pallas_call(
        _epi_stacked_body,
        grid=(N // BN,),
        in_specs=[qspec] * 2 + [
            pl.BlockSpec((4, BN, 1), lambda i: (0, i, 0)),
            pl.BlockSpec((4, D), lambda i: (0, 0)),
            pl.BlockSpec((1, 1), lambda i: (0, 0)),
        ],
        out_specs=[pl.BlockSpec((4, BN, D), lambda i: (0, i, 0))],
        out_shape=[jax.ShapeDtypeStruct((4, N, D), jnp.float32)],
    )(*rq, den3, bst, pa)[0]


def _edge_body(h0, h1, ap_hbm, srcs, dsts,
               rq0, rq1, den_o,
               src_t, dst_t, ap_t, ex_t, denloc,
               gbuf, gbuf2, sbuf, zbuf,
               raw_s, gsem, gsem2):
    c = lax.axis_index("c")
    s = lax.axis_index("s")

    def _zz(i, _):
        for f in range(2):
            zbuf[i, pl.ds(f * 32, 32)] = jnp.zeros((32,), jnp.bfloat16)
        return 0
    lax.fori_loop(0, 157, _zz, 0)

    def _zero_raw():
        for q in range(4):
            pltpu.sync_copy(zbuf, raw_s.at[pl.ds(s * 628 + q * 157, 157)])

    def _branch(bi, _):
        pltpu.sync_copy(srcs.at[bi, s], src_t)
        pltpu.sync_copy(dsts.at[bi, s], dst_t)
        pltpu.sync_copy(ap_hbm.at[bi], ap_t)

        def _zd(i, _2):
            denloc[i] = jnp.zeros((16,), jnp.float32)
            return 0
        lax.fori_loop(0, 640, _zd, 0)
        _zero_raw()

        boff = bi * N

        def _p1(j, _2):
            base = j * TB
            for g in range(8):
                off = base + g * 16
                sv = src_t[pl.ds(off, 16)]
                dv = dst_t[j, pl.ds(g * 16, 16)]
                su = sv - boff
                av = plsc.load_gather(ap_t, [su + su])
                bv = plsc.load_gather(ap_t, [dv + dv + 1])
                al = av + bv
                al = jnp.where(al >= 0.0, al, 0.2 * al)
                e = jnp.exp(al)
                ex_t[pl.ds(off, 16)] = e
                plsc.addupdate_scatter(denloc, [dv >> 4, dv & 15], e)
            return 0
        lax.fori_loop(0, CB, _p1, 0)

        @pl.when(c == 0)
        def _():
            pltpu.sync_copy(denloc, den_o.at[bi, s])

        plsc.subcore_barrier()

        def _p3(h_hbm):
            gb = (gbuf, gbuf2)
            gs = (gsem, gsem2)

            def _start_g(t, k):
                pltpu.async_copy(h_hbm.at[src_t.at[pl.ds(t * TB, TB)]],
                                 gb[k], gs[k])

            def _wait_g(t, k):
                pltpu.make_async_copy(h_hbm.at[src_t.at[pl.ds(t * TB, TB)]],
                                      gb[k], gs[k]).wait()

            def _scale_store(t, k):
                def _sc16(r16, _2):
                    ev16 = ex_t[pl.ds(t * TB + r16 * 16, 16)]
                    for rr in range(16):
                        evb = jnp.broadcast_to(ev16[rr], (16,))
                        r = r16 * 16 + rr
                        for f in range(2):
                            sl = pl.ds(f * 32, 32)
                            va, vb = plsc.unpack(
                                gb[k][r, sl],
                                format=plsc.PackFormat.INTERLEAVED)
                            sbuf[r, sl] = plsc.pack(
                                va * evb, vb * evb,
                                format=plsc.PackFormat.INTERLEAVED)
                    return 0
                lax.fori_loop(0, 8, _sc16, 0)
                pltpu.sync_copy(sbuf, raw_s.at[dst_t.at[t]], add=True)

            _start_g(0, 0)

            def _pair(p, _2):
                t0 = 2 * p
                t1 = t0 + 1
                _start_g(t1, 1)
                _wait_g(t0, 0)
                _scale_store(t0, 0)

                @pl.when(p < CB // 2 - 1)
                def _():
                    _start_g(t0 + 2, 0)
                _wait_g(t1, 1)
                _scale_store(t1, 1)
                return 0
            lax.fori_loop(0, CB // 2, _pair, 0)

        def _wout(rq):
            pltpu.sync_copy(raw_s.at[pl.ds(s * 628, 628)],
                            rq.at[bi, pl.ds(s * 628, 628)])

        @pl.when(c == 0)
        def _():
            _p3(h0)

        @pl.when(c == 1)
        def _():
            _p3(h1)

        plsc.subcore_barrier()

        @pl.when(c == 0)
        def _():
            _wout(rq0)

        @pl.when(c == 1)
        def _():
            _wout(rq1)

        plsc.subcore_barrier()
        return 0

    lax.fori_loop(0, 4, _branch, 0)


def _edge_kernel(h0, h1, ap_pad, srcs, dsts):
    return pl.kernel(
        _edge_body,
        mesh=plsc.VectorSubcoreMesh(core_axis_name="c", subcore_axis_name="s"),
        compiler_params=pltpu.CompilerParams(needs_layout_passes=False,
                                             use_tc_tiling_on_sc=False),
        out_type=[jax.ShapeDtypeStruct((4, NP_, HW), jnp.bfloat16)] * 2
        + [jax.ShapeDtypeStruct((4, NT, 640, 16), jnp.float32)],
        scratch_types=[
            pltpu.VMEM((CHUNK,), jnp.int32),
            pltpu.VMEM((CB, TB), jnp.int32),
            pltpu.VMEM((APW,), jnp.float32),
            pltpu.VMEM((CHUNK,), jnp.float32),
            pltpu.VMEM((640, 16), jnp.float32),
            pltpu.VMEM((TB, HW), jnp.bfloat16),
            pltpu.VMEM((TB, HW), jnp.bfloat16),
            pltpu.VMEM((TB, HW), jnp.bfloat16),
            pltpu.VMEM((157, HW), jnp.bfloat16),
            pltpu.VMEM_SHARED((NP_, HW), jnp.bfloat16),
            pltpu.SemaphoreType.DMA,
            pltpu.SemaphoreType.DMA,
        ],
    )(h0, h1, ap_pad, srcs, dsts)


def kernel(x, ei_g1_pos, ei_g2_pos, ei_g1_neg, ei_g2_neg,
           Wp0, asp0, adp0, bp0, Wp1, asp1, adp1, bp1,
           Wn0, asn0, adn0, bn0, Wn1, asn1, adn1, bn1,
           prelu_a):
    loop = jnp.arange(N, dtype=jnp.int32)
    srcs, dsts = [], []
    for b, ei in enumerate((ei_g1_pos, ei_g2_pos, ei_g1_neg, ei_g2_neg)):
        s_full = jnp.concatenate(
            [ei[0], loop, jnp.zeros((E2P - E2,), jnp.int32)]) + b * N
        d_full = jnp.concatenate(
            [ei[1], loop, jnp.full((E2P - E2,), GHOST, jnp.int32)])
        srcs.append(s_full.reshape(NT, CHUNK))
        dsts.append(d_full.reshape(NT, CB, TB))
    srcs = jnp.stack(srcs)
    dsts = jnp.stack(dsts)

    Wst = jnp.stack([jnp.stack([Wp0, Wp0, Wn0, Wn0]),
                     jnp.stack([Wp1, Wp1, Wn1, Wn1])])
    a0p = jnp.stack([asp0, adp0], axis=1)
    a0n = jnp.stack([asn0, adn0], axis=1)
    a1p = jnp.stack([asp1, adp1], axis=1)
    a1n = jnp.stack([asn1, adn1], axis=1)
    Ast = jnp.stack([jnp.stack([a0p, a0p, a0n, a0n]),
                     jnp.stack([a1p, a1p, a1n, a1n])])
    Bst = jnp.stack([jnp.stack([bp0, bp0, bn0, bn0]),
                     jnp.stack([bp1, bp1, bn1, bn1])])
    pa = prelu_a.reshape(1, 1)

    def layer_step(y, wab):
        w4, a4, b4 = wab
        h0, h1, ap = _dense(y, w4, a4)
        ap_pad = jnp.pad(ap.reshape(4, 2 * N), ((0, 0), (0, APW - 2 * N)))
        out = _edge_kernel(h0.reshape(4 * N, HW), h1.reshape(4 * N, HW),
                           ap_pad, srcs, dsts)
        den3 = out[2].sum(axis=1).reshape(4, NPD, 1)
        return _epilogue_stacked(out[:2], den3, b4, pa), 0

    y0 = jnp.broadcast_to(x[None], (4, N, D))
    yfin, _ = lax.scan(layer_step, y0, (Wst, Ast, Bst))
    return (yfin[0], yfin[1], yfin[2], yfin[3])
